# Initial kernel scaffold; baseline (speedup 1.0000x reference)
import functools

import jax
import jax.numpy as jnp
from jax import lax
from jax.experimental import pallas as pl
from jax.experimental.pallas import tpu as pltpu
from jax.experimental.pallas import tpu_sc as plsc

_NC, _NS, _L = 2, 16, 16
_NW = _NC * _NS

_B, _T, _H, _W = 16, 128, 64, 64
_DIM_S = 256
_WG = _W // _L
_GROUPS = _B * _H * _WG
_GPW = _GROUPS // _NW


@functools.partial(
    pl.kernel,
    out_type=jax.ShapeDtypeStruct((_B, _DIM_S, _H, _W), jnp.int32),
    mesh=plsc.VectorSubcoreMesh(core_axis_name="c", subcore_axis_name="s"),
    scratch_types=[
        pltpu.VMEM((_T, _L), jnp.int32),
        pltpu.VMEM((_DIM_S, _L), jnp.int32),
    ],
)
def _spike_hist(in_hbm, out_hbm, inbuf, hist):
    wid = lax.axis_index("s") * _NC + lax.axis_index("c")
    lanes = lax.iota(jnp.int32, _L)
    ones = jnp.ones((_L,), jnp.int32)
    zeros = jnp.zeros((_L,), jnp.int32)

    def group_body(g, carry):
        gid = wid * _GPW + g
        b = gid // (_H * _WG)
        rem = gid % (_H * _WG)
        h = rem // _WG
        w0 = (rem % _WG) * _L

        pltpu.sync_copy(in_hbm.at[b, :, h, pl.ds(w0, _L)], inbuf)

        @plsc.parallel_loop(0, _DIM_S, unroll=8)
        def _zero(i):
            hist[i, :] = zeros

        def t_body(t, c):
            vals = inbuf[t, :]
            plsc.addupdate_scatter(hist, [vals, lanes], ones)
            return c

        lax.fori_loop(0, _T, t_body, 0, unroll=8)

        pltpu.sync_copy(hist, out_hbm.at[b, :, h, pl.ds(w0, _L)])
        return carry

    lax.fori_loop(0, _GPW, group_body, 0)


def kernel(input, dim_s):
    del dim_s
    return _spike_hist(input)


# speedup vs baseline: 49.4851x; 49.4851x over previous
import functools

import jax
import jax.numpy as jnp
from jax import lax
from jax.experimental import pallas as pl
from jax.experimental.pallas import tpu as pltpu
from jax.experimental.pallas import tpu_sc as plsc

_NC, _NS, _L = 2, 16, 16
_NW = _NC * _NS

_B, _T, _H, _W = 16, 128, 64, 64
_DIM_S = 256
_GROUPS = _B * _H
_GPW = _GROUPS // _NW
_WSUB = _W // _L


@functools.partial(
    pl.kernel,
    out_type=jax.ShapeDtypeStruct((_B * _DIM_S * _H, _W), jnp.int32),
    mesh=plsc.VectorSubcoreMesh(core_axis_name="c", subcore_axis_name="s"),
    scratch_types=[
        pltpu.VMEM((_T, _W), jnp.int32),
        pltpu.VMEM((_DIM_S, _W), jnp.int32),
        pltpu.VMEM((_T,), jnp.int32),
        pltpu.VMEM((_T,), jnp.int32),
        pltpu.VMEM((_T,), jnp.int32),
        pltpu.SemaphoreType.DMA,
    ],
    compiler_params=pltpu.CompilerParams(
        needs_layout_passes=False, use_tc_tiling_on_sc=False),
)
def _spike_hist(in_hbm, out_hbm, inbuf, hist, idx_in, idx_out0, idx_out1, sem):
    wid = lax.axis_index("s") * _NC + lax.axis_index("c")
    lanes = lax.iota(jnp.int32, _L)
    ones = jnp.ones((_L,), jnp.int32)
    zeros = jnp.zeros((_L,), jnp.int32)

    def group_body(g, carry):
        gid = wid * _GPW + g
        b = gid // _H
        h = gid % _H

        in_base = (b * _T) * _H + h
        out_base = (b * _DIM_S) * _H + h

        @plsc.parallel_loop(0, _T, step=_L, unroll=8)
        def _mkidx(i):
            step = (i + lanes) * _H
            idx_in[pl.ds(i, _L)] = in_base + step
            idx_out0[pl.ds(i, _L)] = out_base + step
            idx_out1[pl.ds(i, _L)] = out_base + _T * _H + step

        pltpu.async_copy(in_hbm.at[idx_in], inbuf, sem).wait()

        @plsc.parallel_loop(0, _DIM_S, unroll=4)
        def _zero(i):
            for k in range(_WSUB):
                hist[i, pl.ds(k * _L, _L)] = zeros

        def t_body(t, c):
            for k in range(_WSUB):
                vals = inbuf[t, pl.ds(k * _L, _L)]
                plsc.addupdate_scatter(hist, [vals, lanes + (k * _L)], ones)
            return c

        lax.fori_loop(0, _T, t_body, 0, unroll=4)

        pltpu.async_copy(hist.at[pl.ds(0, _T)], out_hbm.at[idx_out0], sem).wait()
        pltpu.async_copy(hist.at[pl.ds(_T, _T)], out_hbm.at[idx_out1], sem).wait()
        return carry

    lax.fori_loop(0, _GPW, group_body, 0)


def kernel(input, dim_s):
    del dim_s
    out2d = _spike_hist(input.reshape(_B * _T * _H, _W))
    return out2d.reshape(_B, _DIM_S, _H, _W)


# speedup vs baseline: 49.8603x; 1.0076x over previous
import functools

import jax
import jax.numpy as jnp
from jax import lax
from jax.experimental import pallas as pl
from jax.experimental.pallas import tpu as pltpu
from jax.experimental.pallas import tpu_sc as plsc

_NC, _NS, _L = 2, 16, 16
_NW = _NC * _NS

_B, _T, _H, _W = 16, 128, 64, 64
_DIM_S = 256
_GROUPS = _B * _H
_GPW = _GROUPS // _NW
_WSUB = _W // _L


@functools.partial(
    pl.kernel,
    out_type=jax.ShapeDtypeStruct((_B, _DIM_S, _H, _W), jnp.int32),
    mesh=plsc.VectorSubcoreMesh(core_axis_name="c", subcore_axis_name="s"),
    scratch_types=[
        pltpu.VMEM((_T, _W), jnp.int32),
        pltpu.VMEM((_DIM_S, _W), jnp.int32),
        pltpu.SemaphoreType.DMA,
    ],
    compiler_params=pltpu.CompilerParams(
        needs_layout_passes=False, use_tc_tiling_on_sc=False),
)
def _spike_hist(in_hbm, out_hbm, inbuf, hist, sem):
    wid = lax.axis_index("s") * _NC + lax.axis_index("c")
    lanes = lax.iota(jnp.int32, _L)
    ones = jnp.ones((_L,), jnp.int32)
    zeros = jnp.zeros((_L,), jnp.int32)

    def group_body(g, carry):
        gid = wid * _GPW + g
        b = gid // _H
        h = gid % _H

        pltpu.async_copy(in_hbm.at[b, :, h, :], inbuf, sem).wait()

        @plsc.parallel_loop(0, _DIM_S, unroll=4)
        def _zero(i):
            for k in range(_WSUB):
                hist[i, pl.ds(k * _L, _L)] = zeros

        def t_body(t, c):
            for k in range(_WSUB):
                vals = inbuf[t, pl.ds(k * _L, _L)]
                plsc.addupdate_scatter(hist, [vals, lanes + (k * _L)], ones)
            return c

        lax.fori_loop(0, _T, t_body, 0, unroll=4)

        pltpu.async_copy(hist, out_hbm.at[b, :, h, :], sem).wait()
        return carry

    lax.fori_loop(0, _GPW, group_body, 0)


def kernel(input, dim_s):
    del dim_s
    return _spike_hist(input)


# speedup vs baseline: 56.5444x; 1.1341x over previous
import functools

import jax
import jax.numpy as jnp
from jax import lax
from jax.experimental import pallas as pl
from jax.experimental.pallas import tpu as pltpu
from jax.experimental.pallas import tpu_sc as plsc

_NC, _NS, _L = 2, 16, 16
_NW = _NC * _NS

_B, _T, _H, _W = 16, 128, 64, 64
_DIM_S = 256
_HP = 2
_GROUPS = _B * _H // _HP
_GPW = _GROUPS // _NW
_PIX = _HP * _W
_NSUB = _PIX // _L


@functools.partial(
    pl.kernel,
    out_type=jax.ShapeDtypeStruct((_B, _DIM_S, _H, _W), jnp.int32),
    mesh=plsc.VectorSubcoreMesh(core_axis_name="c", subcore_axis_name="s"),
    scratch_types=[
        pltpu.VMEM((2, _T, _HP, _W), jnp.int32),
        pltpu.VMEM((2, _DIM_S, _HP, _W), jnp.int32),
        pltpu.SemaphoreType.DMA,
        pltpu.SemaphoreType.DMA,
        pltpu.SemaphoreType.DMA,
        pltpu.SemaphoreType.DMA,
    ],
    compiler_params=pltpu.CompilerParams(
        needs_layout_passes=False, use_tc_tiling_on_sc=False),
)
def _spike_hist(in_hbm, out_hbm, inbuf, hist, si0, si1, so0, so1):
    wid = lax.axis_index("s") * _NC + lax.axis_index("c")
    lanes = lax.iota(jnp.int32, _L)
    ones = jnp.ones((_L,), jnp.int32)
    zeros = jnp.zeros((_L,), jnp.int32)
    sin = (si0, si1)
    sout = (so0, so1)

    def bh(g):
        gid = wid * _GPW + g
        b = gid // (_H // _HP)
        h = (gid % (_H // _HP)) * _HP
        return b, h

    def start_in(g, slot, sem):
        b, h = bh(g)
        return pltpu.async_copy(
            in_hbm.at[b, :, pl.ds(h, _HP), :], inbuf.at[slot], sem)

    def start_out(g, slot, sem):
        b, h = bh(g)
        return pltpu.async_copy(
            hist.at[slot], out_hbm.at[b, :, pl.ds(h, _HP), :], sem)

    start_in(0, 0, sin[0])
    start_in(1, 1, sin[1])

    def pair_body(g2, carry):
        for slot in range(2):
            g = g2 * 2 + slot

            @pl.when(g2 > 0)
            def _drain():
                b, h = bh(g)
                pltpu.make_async_copy(
                    hist.at[slot], out_hbm.at[b, :, pl.ds(h, _HP), :],
                    sout[slot]).wait()

            @plsc.parallel_loop(0, _DIM_S, unroll=2)
            def _zero(i):
                for k in range(_NSUB):
                    hist[slot, i, k // 4, pl.ds((k % 4) * _L, _L)] = zeros

            b, h = bh(g)
            pltpu.make_async_copy(
                in_hbm.at[b, :, pl.ds(h, _HP), :], inbuf.at[slot],
                sin[slot]).wait()

            hs = hist.at[slot]

            def t_body(t, c):
                for k in range(_NSUB):
                    vals = inbuf[slot, t, k // 4, pl.ds((k % 4) * _L, _L)]
                    plsc.addupdate_scatter(
                        hs, [vals, jnp.full((_L,), k // 4, jnp.int32),
                             lanes + (k % 4) * _L], ones)
                return c

            lax.fori_loop(0, _T, t_body, 0, unroll=2)

            @pl.when(g + 2 < _GPW)
            def _prefetch():
                start_in(g + 2, slot, sin[slot])

            start_out(g, slot, sout[slot])
        return carry

    lax.fori_loop(0, _GPW // 2, pair_body, 0)

    for slot in range(2):
        g = _GPW - 2 + slot
        b, h = bh(g)
        pltpu.make_async_copy(
            hist.at[slot], out_hbm.at[b, :, pl.ds(h, _HP), :], sout[slot]).wait()


def kernel(input, dim_s):
    del dim_s
    return _spike_hist(input)


# speedup vs baseline: 74.3042x; 1.3141x over previous
import functools

import jax
import jax.numpy as jnp
from jax import lax
from jax.experimental import pallas as pl
from jax.experimental.pallas import tpu as pltpu
from jax.experimental.pallas import tpu_sc as plsc

_NC, _NS, _L = 2, 16, 16
_NW = _NC * _NS

_B, _T, _H, _W = 16, 128, 64, 64
_DIM_S = 256
_HP = 2
_GROUPS = _B * _H // _HP
_GPW = _GROUPS // _NW
_PIX = _HP * _W
_NSUB = _PIX // _L


@functools.partial(
    pl.kernel,
    out_type=jax.ShapeDtypeStruct((_B, _DIM_S, _H, _W), jnp.int32),
    mesh=plsc.VectorSubcoreMesh(core_axis_name="c", subcore_axis_name="s"),
    scratch_types=[
        pltpu.VMEM((2, _T, _HP, _W), jnp.int32),
        pltpu.VMEM((2, _DIM_S, _HP, _W), jnp.int32),
        pltpu.SemaphoreType.DMA,
        pltpu.SemaphoreType.DMA,
        pltpu.SemaphoreType.DMA,
        pltpu.SemaphoreType.DMA,
    ],
    compiler_params=pltpu.CompilerParams(
        needs_layout_passes=False, use_tc_tiling_on_sc=False),
)
def _spike_hist(in_hbm, out_hbm, inbuf, hist, si0, si1, so0, so1):
    wid = lax.axis_index("s") * _NC + lax.axis_index("c")
    lanes = lax.iota(jnp.int32, _L)
    ones = jnp.ones((_L,), jnp.int32)
    zeros = jnp.zeros((_L,), jnp.int32)
    sin = (si0, si1)
    sout = (so0, so1)

    def bh(g):
        gid = wid * _GPW + g
        b = gid // (_H // _HP)
        h = (gid % (_H // _HP)) * _HP
        return b, h

    def start_in(g, slot, sem):
        b, h = bh(g)
        return pltpu.async_copy(
            in_hbm.at[b, :, pl.ds(h, _HP), :], inbuf.at[slot], sem)

    def start_out(g, slot, sem):
        b, h = bh(g)
        return pltpu.async_copy(
            hist.at[slot], out_hbm.at[b, :, pl.ds(h, _HP), :], sem)

    start_in(0, 0, sin[0])
    start_in(1, 1, sin[1])

    def pair_body(g2, carry):
        for slot in range(2):
            g = g2 * 2 + slot

            @pl.when(g2 > 0)
            def _drain():
                b, h = bh(g)
                pltpu.make_async_copy(
                    hist.at[slot], out_hbm.at[b, :, pl.ds(h, _HP), :],
                    sout[slot]).wait()

            @plsc.parallel_loop(0, _DIM_S, unroll=2)
            def _zero(i):
                for k in range(_NSUB):
                    hist[slot, i, k // 4, pl.ds((k % 4) * _L, _L)] = zeros

            b, h = bh(g)
            pltpu.make_async_copy(
                in_hbm.at[b, :, pl.ds(h, _HP), :], inbuf.at[slot],
                sin[slot]).wait()

            hs = hist.at[slot]

            @plsc.parallel_loop(0, _T, unroll=4)
            def _t_body(t):
                for k in range(_NSUB):
                    vals = inbuf[slot, t, k // 4, pl.ds((k % 4) * _L, _L)]
                    plsc.addupdate_scatter(
                        hs, [vals, jnp.full((_L,), k // 4, jnp.int32),
                             lanes + (k % 4) * _L], ones)

            @pl.when(g + 2 < _GPW)
            def _prefetch():
                start_in(g + 2, slot, sin[slot])

            start_out(g, slot, sout[slot])
        return carry

    lax.fori_loop(0, _GPW // 2, pair_body, 0)

    for slot in range(2):
        g = _GPW - 2 + slot
        b, h = bh(g)
        pltpu.make_async_copy(
            hist.at[slot], out_hbm.at[b, :, pl.ds(h, _HP), :], sout[slot]).wait()


def kernel(input, dim_s):
    del dim_s
    return _spike_hist(input)
